# baseline (speedup 1.0000x reference)
import functools

import jax
import jax.numpy as jnp
from jax import lax
from jax.experimental import pallas as pl
from jax.experimental.pallas import tpu as pltpu
from jax.experimental.pallas import tpu_sc as plsc

_NUM_EMB = 1000000
_DIM = 32
_CUDA_ROWS = 131072
_BATCH = 16384

_NW = 16
_NB = _BATCH // _NW
_KB = _NB // 128
_NCID = _CUDA_ROWS // _NW
_KC = _NCID // 128
_REPAIR = 6

_L1 = _NUM_EMB + _CUDA_ROWS
_W2 = _CUDA_ROWS + _BATCH
_R2 = 2 * _BATCH


def _body(weight, cache, ids, idx_map, cidx, inv,
          out, last1, win2, rows2x,
          bid, bcpu, bg, biv, bs, bv, bw, dwi, dci,
          bcid, bsv, bl, brid, rows_a, rows_b, sem):
    wid = lax.axis_index("s")
    ibase = wid * _NB
    cbase = wid * _NCID
    iota = lax.iota(jnp.int32, 16)

    for k in range(_KB):
        pltpu.make_async_copy(ids.at[pl.ds(ibase + k * 128, 128)],
                              bid.at[k], sem).start()
    for k in range(_KB):
        pltpu.make_async_copy(ids.at[pl.ds(ibase + k * 128, 128)],
                              bid.at[k], sem).wait()

    def fire_cid(k, c):
        pltpu.make_async_copy(cidx.at[pl.ds(cbase + k * 128, 128)],
                              bcid.at[k], sem).start()
        return c

    def drain_cid(k, c):
        pltpu.make_async_copy(cidx.at[pl.ds(cbase + k * 128, 128)],
                              bcid.at[k], sem).wait()
        return c

    lax.fori_loop(0, _KC, fire_cid, 0)
    lax.fori_loop(0, _KC, drain_cid, 0)

    def gather8(src, idx, dst):
        cps = [pltpu.make_async_copy(src.at[idx.at[k]], dst.at[k], sem)
               for k in range(_KB)]
        for c in cps:
            c.start()
        for c in cps:
            c.wait()

    gather8(idx_map, bid, bcpu)
    gather8(inv, bcpu, bg)

    def fill(buf, nrows, base):
        def row(k, c):
            def col(cc, c2):
                buf[k, pl.ds(cc * 16, 16)] = base + k * 128 + cc * 16 + iota
                return c2
            return lax.fori_loop(0, 8, col, c)
        lax.fori_loop(0, nrows, row, 0)

    fill(biv, _KB, ibase)
    fill(bsv, _KC, cbase)

    def stream64(vals_or_dst, idx, is_scatter):
        def fire(k, c):
            if is_scatter:
                pltpu.make_async_copy(vals_or_dst.at[k],
                                      last1.at[idx.at[k]], sem).start()
            else:
                pltpu.make_async_copy(last1.at[idx.at[k]],
                                      vals_or_dst.at[k], sem).start()
            return c

        def drain(k, c):
            if is_scatter:
                pltpu.make_async_copy(vals_or_dst.at[k],
                                      last1.at[idx.at[k]], sem).wait()
            else:
                pltpu.make_async_copy(last1.at[idx.at[k]],
                                      vals_or_dst.at[k], sem).wait()
            return c

        lax.fori_loop(0, _KC, fire, 0)
        lax.fori_loop(0, _KC, drain, 0)

    def win2_8(vals_or_dst, idx, is_scatter):
        if is_scatter:
            cps = [pltpu.make_async_copy(vals_or_dst.at[k],
                                         win2.at[idx.at[k]], sem)
                   for k in range(_KB)]
        else:
            cps = [pltpu.make_async_copy(win2.at[idx.at[k]],
                                         vals_or_dst.at[k], sem)
                   for k in range(_KB)]
        for c in cps:
            c.start()
        for c in cps:
            c.wait()

    stream64(bsv, bcid, True)
    win2_8(biv, bg, True)
    plsc.subcore_barrier()

    for _ in range(_REPAIR):
        stream64(bl, bcid, False)
        win2_8(bw, bg, False)

        def rrow(k, c):
            def rcol(cc, c2):
                sl = pl.ds(cc * 16, 16)
                lose = bl[k, sl] < bsv[k, sl]
                trash = _NUM_EMB + cbase + k * 128 + cc * 16 + iota
                brid[k, sl] = jnp.where(lose, bcid[k, sl], trash)
                return c2
            return lax.fori_loop(0, 8, rcol, c)
        lax.fori_loop(0, _KC, rrow, 0)

        def wrow(k, c):
            def wcol(cc, c2):
                sl = pl.ds(cc * 16, 16)
                lose = bw[k, sl] < biv[k, sl]
                trash = _CUDA_ROWS + ibase + k * 128 + cc * 16 + iota
                dwi[k, sl] = jnp.where(lose, bg[k, sl], trash)
                return c2
            return lax.fori_loop(0, 8, wcol, c)
        lax.fori_loop(0, _KB, wrow, 0)

        stream64(bsv, brid, True)
        win2_8(biv, dwi, True)
        plsc.subcore_barrier()

    gather8(last1, bcpu, bs)

    def crow(k, c):
        def ccol(cc, c2):
            sl = pl.ds(cc * 16, 16)
            bid[k, sl] = jnp.clip(bs[k, sl], 0, _CUDA_ROWS - 1)
            return c2
        return lax.fori_loop(0, 8, ccol, c)
    lax.fori_loop(0, _KB, crow, 0)

    gather8(cidx, bid, bv)
    win2_8(bw, bg, False)
    gather8(weight, bcpu, rows_a)
    gather8(cache, bid, rows_b)

    def drow(k, c):
        def dcol(cc, c2):
            sl = pl.ds(cc * 16, 16)
            ev = bv[k, sl] == bcpu[k, sl]
            gi = ibase + k * 128 + cc * 16 + iota
            dwi[k, sl] = jnp.where(ev, gi + _BATCH, gi)
            dci[k, sl] = jnp.where(ev, gi, gi + _BATCH)
            return c2
        return lax.fori_loop(0, 8, dcol, c)
    lax.fori_loop(0, _KB, drow, 0)

    cps = ([pltpu.make_async_copy(rows_a.at[k], rows2x.at[dwi.at[k]], sem)
            for k in range(_KB)] +
           [pltpu.make_async_copy(rows_b.at[k], rows2x.at[dci.at[k]], sem)
            for k in range(_KB)])
    for c in cps:
        c.start()
    for c in cps:
        c.wait()
    plsc.subcore_barrier()

    gather8(rows2x, bw, rows_a)
    for k in range(_KB):
        pltpu.make_async_copy(rows_a.at[k],
                              out.at[pl.ds(ibase + k * 128, 128)], sem).start()
    for k in range(_KB):
        pltpu.make_async_copy(rows_a.at[k],
                              out.at[pl.ds(ibase + k * 128, 128)], sem).wait()


@jax.jit
def _run(weight, cache, ids, idx_map, cidx, inv):
    f = pl.kernel(
        _body,
        out_type=[
            jax.ShapeDtypeStruct((_BATCH, _DIM), jnp.float32),
            jax.ShapeDtypeStruct((_L1,), jnp.int32),
            jax.ShapeDtypeStruct((_W2,), jnp.int32),
            jax.ShapeDtypeStruct((_R2, _DIM), jnp.float32),
        ],
        mesh=plsc.VectorSubcoreMesh(core_axis_name="c", subcore_axis_name="s",
                                    num_cores=1),
        compiler_params=pltpu.CompilerParams(use_tc_tiling_on_sc=False),
        scratch_types=[
            pltpu.VMEM((_KB, 128), jnp.int32),
            pltpu.VMEM((_KB, 128), jnp.int32),
            pltpu.VMEM((_KB, 128), jnp.int32),
            pltpu.VMEM((_KB, 128), jnp.int32),
            pltpu.VMEM((_KB, 128), jnp.int32),
            pltpu.VMEM((_KB, 128), jnp.int32),
            pltpu.VMEM((_KB, 128), jnp.int32),
            pltpu.VMEM((_KB, 128), jnp.int32),
            pltpu.VMEM((_KB, 128), jnp.int32),
            pltpu.VMEM((_KC, 128), jnp.int32),
            pltpu.VMEM((_KC, 128), jnp.int32),
            pltpu.VMEM((_KC, 128), jnp.int32),
            pltpu.VMEM((_KC, 128), jnp.int32),
            pltpu.VMEM((_KB, 128, _DIM), jnp.float32),
            pltpu.VMEM((_KB, 128, _DIM), jnp.float32),
            pltpu.SemaphoreType.DMA,
        ],
    )
    o, _, _, _ = f(weight, cache, ids, idx_map, cidx, inv)
    return o


def kernel(weight, cuda_cached_weight, ids, idx_map, cached_idx_map, inverted_cached_idx):
    return _run(weight, cuda_cached_weight, ids, idx_map,
                cached_idx_map, inverted_cached_idx)


# speedup vs baseline: 4.0806x; 4.0806x over previous
import functools

import jax
import jax.numpy as jnp
from jax import lax
from jax.experimental import pallas as pl
from jax.experimental.pallas import tpu as pltpu
from jax.experimental.pallas import tpu_sc as plsc

_NUM_EMB = 1000000
_DIM = 32
_CUDA_ROWS = 131072
_BATCH = 16384

_NW = 16
_NB = _BATCH // _NW
_KB = _NB // 128
_NCID = _CUDA_ROWS // _NW
_KC = _NCID // 128
_REPAIR = 0

_L1 = _NUM_EMB + _CUDA_ROWS
_W2 = _CUDA_ROWS + _BATCH
_R2 = 2 * _BATCH


def _body(weight, cache, ids, idx_map, cidx, inv,
          out, last1, win2, rows2x,
          bid, bcpu, bg, biv, bs, bv, bw, dwi, dci,
          bcid, bsv, bl, brid, rows_a, rows_b, sem):
    wid = lax.axis_index("s")
    ibase = wid * _NB
    cbase = wid * _NCID
    iota = lax.iota(jnp.int32, 16)

    for k in range(_KB):
        pltpu.make_async_copy(ids.at[pl.ds(ibase + k * 128, 128)],
                              bid.at[k], sem).start()
    for k in range(_KB):
        pltpu.make_async_copy(ids.at[pl.ds(ibase + k * 128, 128)],
                              bid.at[k], sem).wait()

    def fire_cid(k, c):
        pltpu.make_async_copy(cidx.at[pl.ds(cbase + k * 128, 128)],
                              bcid.at[k], sem).start()
        return c

    def drain_cid(k, c):
        pltpu.make_async_copy(cidx.at[pl.ds(cbase + k * 128, 128)],
                              bcid.at[k], sem).wait()
        return c

    lax.fori_loop(0, _KC, fire_cid, 0)
    lax.fori_loop(0, _KC, drain_cid, 0)

    def gather8(src, idx, dst):
        cps = [pltpu.make_async_copy(src.at[idx.at[k]], dst.at[k], sem)
               for k in range(_KB)]
        for c in cps:
            c.start()
        for c in cps:
            c.wait()

    gather8(idx_map, bid, bcpu)
    gather8(inv, bcpu, bg)

    def fill(buf, nrows, base):
        def row(k, c):
            def col(cc, c2):
                buf[k, pl.ds(cc * 16, 16)] = base + k * 128 + cc * 16 + iota
                return c2
            return lax.fori_loop(0, 8, col, c)
        lax.fori_loop(0, nrows, row, 0)

    fill(biv, _KB, ibase)
    fill(bsv, _KC, cbase)

    def stream64(vals_or_dst, idx, is_scatter):
        def fire(k, c):
            if is_scatter:
                pltpu.make_async_copy(vals_or_dst.at[k],
                                      last1.at[idx.at[k]], sem).start()
            else:
                pltpu.make_async_copy(last1.at[idx.at[k]],
                                      vals_or_dst.at[k], sem).start()
            return c

        def drain(k, c):
            if is_scatter:
                pltpu.make_async_copy(vals_or_dst.at[k],
                                      last1.at[idx.at[k]], sem).wait()
            else:
                pltpu.make_async_copy(last1.at[idx.at[k]],
                                      vals_or_dst.at[k], sem).wait()
            return c

        lax.fori_loop(0, _KC, fire, 0)
        lax.fori_loop(0, _KC, drain, 0)

    def win2_8(vals_or_dst, idx, is_scatter):
        if is_scatter:
            cps = [pltpu.make_async_copy(vals_or_dst.at[k],
                                         win2.at[idx.at[k]], sem)
                   for k in range(_KB)]
        else:
            cps = [pltpu.make_async_copy(win2.at[idx.at[k]],
                                         vals_or_dst.at[k], sem)
                   for k in range(_KB)]
        for c in cps:
            c.start()
        for c in cps:
            c.wait()

    stream64(bsv, bcid, True)
    win2_8(biv, bg, True)
    plsc.subcore_barrier()

    for _ in range(_REPAIR):
        stream64(bl, bcid, False)
        win2_8(bw, bg, False)

        def rrow(k, c):
            def rcol(cc, c2):
                sl = pl.ds(cc * 16, 16)
                lose = bl[k, sl] < bsv[k, sl]
                trash = _NUM_EMB + cbase + k * 128 + cc * 16 + iota
                brid[k, sl] = jnp.where(lose, bcid[k, sl], trash)
                return c2
            return lax.fori_loop(0, 8, rcol, c)
        lax.fori_loop(0, _KC, rrow, 0)

        def wrow(k, c):
            def wcol(cc, c2):
                sl = pl.ds(cc * 16, 16)
                lose = bw[k, sl] < biv[k, sl]
                trash = _CUDA_ROWS + ibase + k * 128 + cc * 16 + iota
                dwi[k, sl] = jnp.where(lose, bg[k, sl], trash)
                return c2
            return lax.fori_loop(0, 8, wcol, c)
        lax.fori_loop(0, _KB, wrow, 0)

        stream64(bsv, brid, True)
        win2_8(biv, dwi, True)
        plsc.subcore_barrier()

    gather8(last1, bcpu, bs)

    def crow(k, c):
        def ccol(cc, c2):
            sl = pl.ds(cc * 16, 16)
            bid[k, sl] = jnp.clip(bs[k, sl], 0, _CUDA_ROWS - 1)
            return c2
        return lax.fori_loop(0, 8, ccol, c)
    lax.fori_loop(0, _KB, crow, 0)

    gather8(cidx, bid, bv)
    win2_8(bw, bg, False)
    gather8(weight, bcpu, rows_a)
    gather8(cache, bid, rows_b)

    def drow(k, c):
        def dcol(cc, c2):
            sl = pl.ds(cc * 16, 16)
            ev = bv[k, sl] == bcpu[k, sl]
            gi = ibase + k * 128 + cc * 16 + iota
            dwi[k, sl] = jnp.where(ev, gi + _BATCH, gi)
            dci[k, sl] = jnp.where(ev, gi, gi + _BATCH)
            return c2
        return lax.fori_loop(0, 8, dcol, c)
    lax.fori_loop(0, _KB, drow, 0)

    cps = ([pltpu.make_async_copy(rows_a.at[k], rows2x.at[dwi.at[k]], sem)
            for k in range(_KB)] +
           [pltpu.make_async_copy(rows_b.at[k], rows2x.at[dci.at[k]], sem)
            for k in range(_KB)])
    for c in cps:
        c.start()
    for c in cps:
        c.wait()
    plsc.subcore_barrier()

    gather8(rows2x, bw, rows_a)
    for k in range(_KB):
        pltpu.make_async_copy(rows_a.at[k],
                              out.at[pl.ds(ibase + k * 128, 128)], sem).start()
    for k in range(_KB):
        pltpu.make_async_copy(rows_a.at[k],
                              out.at[pl.ds(ibase + k * 128, 128)], sem).wait()


@jax.jit
def _run(weight, cache, ids, idx_map, cidx, inv):
    f = pl.kernel(
        _body,
        out_type=[
            jax.ShapeDtypeStruct((_BATCH, _DIM), jnp.float32),
            jax.ShapeDtypeStruct((_L1,), jnp.int32),
            jax.ShapeDtypeStruct((_W2,), jnp.int32),
            jax.ShapeDtypeStruct((_R2, _DIM), jnp.float32),
        ],
        mesh=plsc.VectorSubcoreMesh(core_axis_name="c", subcore_axis_name="s",
                                    num_cores=1),
        compiler_params=pltpu.CompilerParams(use_tc_tiling_on_sc=False),
        scratch_types=[
            pltpu.VMEM((_KB, 128), jnp.int32),
            pltpu.VMEM((_KB, 128), jnp.int32),
            pltpu.VMEM((_KB, 128), jnp.int32),
            pltpu.VMEM((_KB, 128), jnp.int32),
            pltpu.VMEM((_KB, 128), jnp.int32),
            pltpu.VMEM((_KB, 128), jnp.int32),
            pltpu.VMEM((_KB, 128), jnp.int32),
            pltpu.VMEM((_KB, 128), jnp.int32),
            pltpu.VMEM((_KB, 128), jnp.int32),
            pltpu.VMEM((_KC, 128), jnp.int32),
            pltpu.VMEM((_KC, 128), jnp.int32),
            pltpu.VMEM((_KC, 128), jnp.int32),
            pltpu.VMEM((_KC, 128), jnp.int32),
            pltpu.VMEM((_KB, 128, _DIM), jnp.float32),
            pltpu.VMEM((_KB, 128, _DIM), jnp.float32),
            pltpu.SemaphoreType.DMA,
        ],
    )
    o, _, _, _ = f(weight, cache, ids, idx_map, cidx, inv)
    return o


def kernel(weight, cuda_cached_weight, ids, idx_map, cached_idx_map, inverted_cached_idx):
    return _run(weight, cuda_cached_weight, ids, idx_map,
                cached_idx_map, inverted_cached_idx)
